# baseline (speedup 1.0000x reference)
import functools

import jax
import jax.numpy as jnp
from jax import lax
from jax.experimental import pallas as pl
from jax.experimental.pallas import tpu as pltpu
from jax.experimental.pallas import tpu_sc as plsc

_NC = 2
_NS = 16
_L = 16
_NW = _NC * _NS

_CH = 80
_NBUF = 4
_IBUF = 3


@functools.partial(jax.jit, static_argnames=("n", "d"))
def _scores(wpad, embed, b, *, n, d):
    def body(w_ref, emb_ref, b_ref, out_ref):
        s = lax.dot_general(
            w_ref[...], emb_ref[...],
            (((0,), (1,)), ((), ())),
            preferred_element_type=jnp.float32,
        )
        rowid = lax.broadcasted_iota(jnp.int32, (8, n), 0)
        out_ref[...] = s + jnp.where(rowid == 0, b_ref[0], 0.0)

    return pl.pallas_call(
        body,
        out_shape=jax.ShapeDtypeStruct((8, n), jnp.float32),
    )(wpad, embed, b)


@functools.partial(jax.jit, static_argnames=("n", "d", "e"))
def _sc_edge_aggregate(s1, s2, row, col, x, *, n, d, e):
    ch = _CH
    epw = e // _NW
    nch = epw // ch
    br = 80
    nblk = n // br
    tpb = (nblk + _NS - 1) // _NS
    assert epw * _NW == e and nch * ch == epw and nblk * br == n
    nsteps = nch // _NBUF
    tail = list(range(nsteps * _NBUF, nch))
    assert nsteps >= 3 and len(tail) == nch % _NBUF
    bch = _NBUF * ch
    ltail = len(tail) * ch

    mesh = plsc.VectorSubcoreMesh(core_axis_name="c", subcore_axis_name="s")

    @functools.partial(
        pl.kernel,
        out_type=jax.ShapeDtypeStruct((_NC * n, d), jnp.float32),
        mesh=mesh,
        compiler_params=pltpu.CompilerParams(needs_layout_passes=False),
        scratch_types=[
            pltpu.VMEM((_IBUF * bch,), jnp.int32),
            pltpu.VMEM((_IBUF * bch,), jnp.int32),
            [pltpu.VMEM((ch,), jnp.int32)] * _NBUF,
            [pltpu.VMEM((ch, d), jnp.float32)] * _NBUF,
            [pltpu.VMEM((ch,), jnp.float32)] * _NBUF,
            [pltpu.VMEM((ch,), jnp.float32)] * _NBUF,
            pltpu.VMEM_SHARED((n, d), jnp.float32),
            pltpu.SemaphoreType.DMA,
            [pltpu.SemaphoreType.DMA] * _NBUF,
            [pltpu.SemaphoreType.DMA] * _NBUF,
        ],
    )
    def k(s1_hbm, s2_hbm, row_hbm, col_hbm, x_hbm, out_hbm,
          rowb, colb, colv, xbufs, s1g, s2g, acc, bsem, gsems, ssems):
        cid = lax.axis_index("c")
        sid = lax.axis_index("s")
        wid = cid * _NS + sid
        ebase = wid * epw

        zero16 = jnp.zeros((_L,), jnp.float32)
        zsrc = xbufs[0]

        def zrow(i, carry):
            for j in range(d // _L):
                zsrc[i, pl.ds(j * _L, _L)] = zero16
            return carry

        lax.fori_loop(0, ch, zrow, 0)

        def zblk(t, carry):
            blk = sid + t * _NS

            @pl.when(blk < nblk)
            def _():
                pltpu.async_copy(zsrc, acc.at[pl.ds(blk * br, br)], bsem)

            return carry

        def zdrain(t, carry):
            blk = sid + t * _NS

            @pl.when(blk < nblk)
            def _():
                pltpu.make_async_copy(zsrc, acc.at[pl.ds(0, br)], bsem).wait()

            return carry

        lax.fori_loop(0, tpb, zblk, 0)
        lax.fori_loop(0, tpb, zdrain, 0)
        plsc.subcore_barrier()

        def blk_copy(b, bs, size):
            off = ebase + b * bch
            pltpu.async_copy(row_hbm.at[pl.ds(off, size)],
                             rowb.at[pl.ds(bs * bch, size)], bsem)
            pltpu.async_copy(col_hbm.at[pl.ds(off, size)],
                             colb.at[pl.ds(bs * bch, size)], bsem)

        def blk_drain(size):
            pltpu.make_async_copy(row_hbm.at[pl.ds(0, size)],
                                  rowb.at[pl.ds(0, size)], bsem).wait()
            pltpu.make_async_copy(col_hbm.at[pl.ds(0, size)],
                                  colb.at[pl.ds(0, size)], bsem).wait()

        def gathers_start(bs, kk, s):
            ridx = rowb.at[pl.ds(bs * bch + kk * ch, ch)]
            cidx = colb.at[pl.ds(bs * bch + kk * ch, ch)]
            pltpu.async_copy(x_hbm.at[ridx], xbufs[s], gsems[s])
            pltpu.async_copy(s1_hbm.at[ridx], s1g[s], gsems[s])
            pltpu.async_copy(s2_hbm.at[cidx], s2g[s], gsems[s])

        def gathers_drain(s):
            pltpu.make_async_copy(x_hbm.at[pl.ds(0, ch)], xbufs[s],
                                  gsems[s]).wait()
            pltpu.make_async_copy(s1_hbm.at[pl.ds(0, ch)], s1g[s],
                                  gsems[s]).wait()
            pltpu.make_async_copy(s2_hbm.at[pl.ds(0, ch)], s2g[s],
                                  gsems[s]).wait()

        def scatter_start(s):
            pltpu.async_copy(xbufs[s], acc.at[colv[s]], ssems[s], add=True)

        def scatter_drain(s):
            pltpu.make_async_copy(x_hbm.at[pl.ds(0, ch)], xbufs[s],
                                  ssems[s]).wait()

        def compute(bs, kk, s):
            xr, g1, g2 = xbufs[s], s1g[s], s2g[s]
            for j in range(ch // _L):
                colv[s][pl.ds(j * _L, _L)] = (
                    colb[pl.ds(bs * bch + kk * ch + j * _L, _L)])
            for j in range(ch // _L):
                v = g1[pl.ds(j * _L, _L)] + g2[pl.ds(j * _L, _L)]
                g1[pl.ds(j * _L, _L)] = 1.0 / (1.0 + jnp.exp(-v))

            def mul4(q, vidx):
                i0 = q * 4
                for r in range(4):
                    gi = plsc.load_gather(g1, [vidx])
                    vidx = vidx + 1
                    for j in range(d // _L):
                        xr[i0 + r, pl.ds(j * _L, _L)] = (
                            xr[i0 + r, pl.ds(j * _L, _L)] * gi)
                return vidx

            lax.fori_loop(0, ch // 4, mul4, jnp.zeros((_L,), jnp.int32))

        blk_copy(0, 0, bch)
        blk_copy(1, 1, bch)
        blk_drain(bch)
        gathers_start(0, 0, 0)
        gathers_start(0, 1, 1)

        def step(t, carry):
            bs0 = lax.rem(t, _IBUF)
            bs1 = lax.rem(t + 1, _IBUF)
            bs2 = lax.rem(t + 2, _IBUF)
            for i in range(_NBUF):
                c = t * _NBUF + i
                sp2 = (i + 2) % _NBUF

                if i >= 2:
                    scatter_drain(i - 2)
                else:

                    @pl.when(t > 0)
                    def _():
                        scatter_drain((i - 2) % _NBUF)

                if i == 2:

                    @pl.when(t + 1 < nsteps)
                    def _():
                        blk_drain(bch)

                    if tail:

                        @pl.when(t + 1 == nsteps)
                        def _():
                            blk_drain(ltail)

                    @pl.when(t + 2 < nsteps)
                    def _():
                        blk_copy(t + 2, bs2, bch)

                    if tail:

                        @pl.when(t + 2 == nsteps)
                        def _():
                            blk_copy(t + 2, bs2, ltail)

                bsg = bs0 if i < 2 else bs1

                @pl.when(c + 2 < nch)
                def _():
                    gathers_start(bsg, (i + 2) % _NBUF, sp2)

                gathers_drain(i)
                compute(bs0, i, i)
                scatter_start(i)
            return carry

        lax.fori_loop(0, nsteps, step, 0)

        pending = [(_NBUF - 2) % _NBUF, (_NBUF - 1) % _NBUF]
        for c in tail:
            s = c % _NBUF
            gathers_drain(s)
            compute(nsteps % _IBUF, c % _NBUF, s)
            scatter_start(s)
            pending.append(s)
        for s in pending:
            scatter_drain(s)

        plsc.subcore_barrier()

        def eblk(t, carry):
            blk = sid + t * _NS

            @pl.when(blk < nblk)
            def _():
                pltpu.async_copy(acc.at[pl.ds(blk * br, br)],
                                 out_hbm.at[pl.ds(cid * n + blk * br, br)],
                                 bsem)

            return carry

        def edrain(t, carry):
            blk = sid + t * _NS

            @pl.when(blk < nblk)
            def _():
                pltpu.make_async_copy(acc.at[pl.ds(0, br)],
                                      out_hbm.at[pl.ds(0, br)], bsem).wait()

            return carry

        lax.fori_loop(0, tpb, eblk, 0)
        lax.fori_loop(0, tpb, edrain, 0)

    return k(s1, s2, row, col, x)


@functools.partial(jax.jit, static_argnames=("n", "d"))
def _combine(parts, *, n, d):
    def body(p_ref, o_ref):
        o_ref[...] = p_ref[0] + p_ref[1]

    return pl.pallas_call(
        body,
        out_shape=jax.ShapeDtypeStruct((n, d), jnp.float32),
    )(parts)


@jax.jit
def kernel(x, embed, edge_index, new_edge_index, label, tmp, W, b):
    n, d = x.shape
    e = edge_index.shape[1]
    row = edge_index[0].astype(jnp.int32)
    col = edge_index[1].astype(jnp.int32)
    w = W.astype(jnp.float32).reshape(2 * d)
    wpad = jnp.zeros((d, 8), jnp.float32)
    wpad = wpad.at[:, 0].set(w[:d]).at[:, 1].set(w[d:])

    s8 = _scores(wpad, embed.astype(jnp.float32), b.astype(jnp.float32),
                 n=n, d=d)
    parts = _sc_edge_aggregate(s8[0], s8[1], row, col,
                               x.astype(jnp.float32), n=n, d=d, e=e)
    return _combine(parts.reshape(_NC, n, d), n=n, d=d)


# speedup vs baseline: 1.0268x; 1.0268x over previous
import functools

import jax
import jax.numpy as jnp
from jax import lax
from jax.experimental import pallas as pl
from jax.experimental.pallas import tpu as pltpu
from jax.experimental.pallas import tpu_sc as plsc

_NC = 2
_NS = 16
_L = 16
_NW = _NC * _NS

_CH = 80
_NBUF = 4
_IBUF = 3


@functools.partial(jax.jit, static_argnames=("n", "d"))
def _scores(wpad, embed, b, *, n, d):
    def body(w_ref, emb_ref, b_ref, out_ref):
        s = lax.dot_general(
            w_ref[...], emb_ref[...],
            (((0,), (1,)), ((), ())),
            preferred_element_type=jnp.float32,
        )
        rowid = lax.broadcasted_iota(jnp.int32, (8, n), 0)
        out_ref[...] = s + jnp.where(rowid == 0, b_ref[0], 0.0)

    return pl.pallas_call(
        body,
        out_shape=jax.ShapeDtypeStruct((8, n), jnp.float32),
    )(wpad, embed, b)


@functools.partial(jax.jit, static_argnames=("n", "d", "e"))
def _sc_edge_aggregate(s1, s2, row, col, x, *, n, d, e):
    ch = _CH
    epw = e // _NW
    nch = epw // ch
    br = 80
    nblk = n // br
    tpb = (nblk + _NS - 1) // _NS
    assert epw * _NW == e and nch * ch == epw and nblk * br == n
    nsteps = nch // _NBUF
    tail = list(range(nsteps * _NBUF, nch))
    assert nsteps >= 3 and len(tail) == nch % _NBUF
    bch = _NBUF * ch
    ltail = len(tail) * ch

    mesh = plsc.VectorSubcoreMesh(core_axis_name="c", subcore_axis_name="s")

    @functools.partial(
        pl.kernel,
        out_type=jax.ShapeDtypeStruct((_NC * n, d), jnp.float32),
        mesh=mesh,
        compiler_params=pltpu.CompilerParams(needs_layout_passes=False),
        scratch_types=[
            pltpu.VMEM((_IBUF * bch,), jnp.int32),
            pltpu.VMEM((_IBUF * bch,), jnp.int32),
            [pltpu.VMEM((ch,), jnp.int32)] * _NBUF,
            [pltpu.VMEM((ch, d), jnp.float32)] * _NBUF,
            [pltpu.VMEM((ch,), jnp.float32)] * _NBUF,
            [pltpu.VMEM((ch,), jnp.float32)] * _NBUF,
            pltpu.VMEM_SHARED((n, d), jnp.float32),
            pltpu.SemaphoreType.DMA,
            [pltpu.SemaphoreType.DMA] * _NBUF,
            [pltpu.SemaphoreType.DMA] * _NBUF,
        ],
    )
    def k(s1_hbm, s2_hbm, row_hbm, col_hbm, x_hbm, out_hbm,
          rowb, colb, colv, xbufs, s1g, s2g, acc, bsem, gsems, ssems):
        cid = lax.axis_index("c")
        sid = lax.axis_index("s")
        wid = cid * _NS + sid
        ebase = wid * epw

        zero16 = jnp.zeros((_L,), jnp.float32)
        zsrc = xbufs[0]

        def zrow(i, carry):
            for j in range(d // _L):
                zsrc[i, pl.ds(j * _L, _L)] = zero16
            return carry

        lax.fori_loop(0, ch, zrow, 0)

        def zblk(t, carry):
            blk = sid + t * _NS

            @pl.when(blk < nblk)
            def _():
                pltpu.async_copy(zsrc, acc.at[pl.ds(blk * br, br)], bsem)

            return carry

        def zdrain(t, carry):
            blk = sid + t * _NS

            @pl.when(blk < nblk)
            def _():
                pltpu.make_async_copy(zsrc, acc.at[pl.ds(0, br)], bsem).wait()

            return carry

        lax.fori_loop(0, tpb, zblk, 0)
        lax.fori_loop(0, tpb, zdrain, 0)
        plsc.subcore_barrier()

        def blk_copy(b, bs, size):
            off = ebase + b * bch
            pltpu.async_copy(row_hbm.at[pl.ds(off, size)],
                             rowb.at[pl.ds(bs * bch, size)], bsem)
            pltpu.async_copy(col_hbm.at[pl.ds(off, size)],
                             colb.at[pl.ds(bs * bch, size)], bsem)

        def blk_drain(size):
            pltpu.make_async_copy(row_hbm.at[pl.ds(0, size)],
                                  rowb.at[pl.ds(0, size)], bsem).wait()
            pltpu.make_async_copy(col_hbm.at[pl.ds(0, size)],
                                  colb.at[pl.ds(0, size)], bsem).wait()

        def gathers_start(bs, kk, s):
            ridx = rowb.at[pl.ds(bs * bch + kk * ch, ch)]
            cidx = colb.at[pl.ds(bs * bch + kk * ch, ch)]
            pltpu.async_copy(x_hbm.at[ridx], xbufs[s], gsems[s])
            pltpu.async_copy(s1_hbm.at[ridx], s1g[s], gsems[s])
            pltpu.async_copy(s2_hbm.at[cidx], s2g[s], gsems[s])

        def gathers_drain(s):
            pltpu.make_async_copy(x_hbm.at[pl.ds(0, ch)], xbufs[s],
                                  gsems[s]).wait()
            pltpu.make_async_copy(s1_hbm.at[pl.ds(0, ch)], s1g[s],
                                  gsems[s]).wait()
            pltpu.make_async_copy(s2_hbm.at[pl.ds(0, ch)], s2g[s],
                                  gsems[s]).wait()

        def scatter_start(s):
            pltpu.async_copy(xbufs[s], acc.at[colv[s]], ssems[s], add=True)

        def scatter_drain(s):
            pltpu.make_async_copy(x_hbm.at[pl.ds(0, ch)], xbufs[s],
                                  ssems[s]).wait()

        def compute(bs, kk, s):
            xr, g1, g2 = xbufs[s], s1g[s], s2g[s]
            for j in range(ch // _L):
                colv[s][pl.ds(j * _L, _L)] = (
                    colb[pl.ds(bs * bch + kk * ch + j * _L, _L)])
            for j in range(ch // _L):
                v = g1[pl.ds(j * _L, _L)] + g2[pl.ds(j * _L, _L)]
                g1[pl.ds(j * _L, _L)] = 1.0 / (1.0 + jnp.exp(-v))

            def mul8(q, vidx):
                i0 = q * 8
                gis = []
                for r in range(8):
                    gis.append(plsc.load_gather(g1, [vidx]))
                    vidx = vidx + 1
                for r in range(8):
                    for j in range(d // _L):
                        xr[i0 + r, pl.ds(j * _L, _L)] = (
                            xr[i0 + r, pl.ds(j * _L, _L)] * gis[r])
                return vidx

            lax.fori_loop(0, ch // 8, mul8, jnp.zeros((_L,), jnp.int32))

        blk_copy(0, 0, bch)
        blk_copy(1, 1, bch)
        blk_drain(bch)
        gathers_start(0, 0, 0)
        gathers_start(0, 1, 1)

        def step(t, carry):
            bs0 = lax.rem(t, _IBUF)
            bs1 = lax.rem(t + 1, _IBUF)
            bs2 = lax.rem(t + 2, _IBUF)
            for i in range(_NBUF):
                c = t * _NBUF + i
                sp2 = (i + 2) % _NBUF

                if i >= 2:
                    scatter_drain(i - 2)
                else:

                    @pl.when(t > 0)
                    def _():
                        scatter_drain((i - 2) % _NBUF)

                if i == 2:

                    @pl.when(t + 1 < nsteps)
                    def _():
                        blk_drain(bch)

                    if tail:

                        @pl.when(t + 1 == nsteps)
                        def _():
                            blk_drain(ltail)

                    @pl.when(t + 2 < nsteps)
                    def _():
                        blk_copy(t + 2, bs2, bch)

                    if tail:

                        @pl.when(t + 2 == nsteps)
                        def _():
                            blk_copy(t + 2, bs2, ltail)

                bsg = bs0 if i < 2 else bs1

                @pl.when(c + 2 < nch)
                def _():
                    gathers_start(bsg, (i + 2) % _NBUF, sp2)

                gathers_drain(i)
                compute(bs0, i, i)
                scatter_start(i)
            return carry

        lax.fori_loop(0, nsteps, step, 0)

        pending = [(_NBUF - 2) % _NBUF, (_NBUF - 1) % _NBUF]
        for c in tail:
            s = c % _NBUF
            gathers_drain(s)
            compute(nsteps % _IBUF, c % _NBUF, s)
            scatter_start(s)
            pending.append(s)
        for s in pending:
            scatter_drain(s)

        plsc.subcore_barrier()

        def eblk(t, carry):
            blk = sid + t * _NS

            @pl.when(blk < nblk)
            def _():
                pltpu.async_copy(acc.at[pl.ds(blk * br, br)],
                                 out_hbm.at[pl.ds(cid * n + blk * br, br)],
                                 bsem)

            return carry

        def edrain(t, carry):
            blk = sid + t * _NS

            @pl.when(blk < nblk)
            def _():
                pltpu.make_async_copy(acc.at[pl.ds(0, br)],
                                      out_hbm.at[pl.ds(0, br)], bsem).wait()

            return carry

        lax.fori_loop(0, tpb, eblk, 0)
        lax.fori_loop(0, tpb, edrain, 0)

    return k(s1, s2, row, col, x)


@functools.partial(jax.jit, static_argnames=("n", "d"))
def _combine(parts, *, n, d):
    def body(p_ref, o_ref):
        o_ref[...] = p_ref[0] + p_ref[1]

    return pl.pallas_call(
        body,
        out_shape=jax.ShapeDtypeStruct((n, d), jnp.float32),
    )(parts)


@jax.jit
def kernel(x, embed, edge_index, new_edge_index, label, tmp, W, b):
    n, d = x.shape
    e = edge_index.shape[1]
    row = edge_index[0].astype(jnp.int32)
    col = edge_index[1].astype(jnp.int32)
    w = W.astype(jnp.float32).reshape(2 * d)
    wpad = jnp.zeros((d, 8), jnp.float32)
    wpad = wpad.at[:, 0].set(w[:d]).at[:, 1].set(w[d:])

    s8 = _scores(wpad, embed.astype(jnp.float32), b.astype(jnp.float32),
                 n=n, d=d)
    parts = _sc_edge_aggregate(s8[0], s8[1], row, col,
                               x.astype(jnp.float32), n=n, d=d, e=e)
    return _combine(parts.reshape(_NC, n, d), n=n, d=d)


# speedup vs baseline: 1.0417x; 1.0145x over previous
import functools

import jax
import jax.numpy as jnp
from jax import lax
from jax.experimental import pallas as pl
from jax.experimental.pallas import tpu as pltpu
from jax.experimental.pallas import tpu_sc as plsc

_NC = 2
_NS = 16
_L = 16
_NW = _NC * _NS

_CH = 80
_NBUF = 4
_IBUF = 3


@functools.partial(jax.jit, static_argnames=("n", "d"))
def _scores(wpad, embed, b, *, n, d):
    def body(w_ref, emb_ref, b_ref, out_ref):
        s = lax.dot_general(
            w_ref[...], emb_ref[...],
            (((0,), (1,)), ((), ())),
            preferred_element_type=jnp.float32,
        )
        rowid = lax.broadcasted_iota(jnp.int32, (8, n), 0)
        out_ref[...] = s + jnp.where(rowid == 0, b_ref[0], 0.0)

    return pl.pallas_call(
        body,
        out_shape=jax.ShapeDtypeStruct((8, n), jnp.float32),
    )(wpad, embed, b)


@functools.partial(jax.jit, static_argnames=("n", "d", "e"))
def _sc_edge_aggregate(s1, s2, row, col, x, *, n, d, e):
    ch = _CH
    epw = e // _NW
    nch = epw // ch
    br = 80
    nblk = n // br
    tpb = (nblk + _NS - 1) // _NS
    assert epw * _NW == e and nch * ch == epw and nblk * br == n
    nsteps = nch // _NBUF
    tail = list(range(nsteps * _NBUF, nch))
    assert nsteps >= 3 and len(tail) == nch % _NBUF
    bch = _NBUF * ch
    ltail = len(tail) * ch

    mesh = plsc.VectorSubcoreMesh(core_axis_name="c", subcore_axis_name="s")

    @functools.partial(
        pl.kernel,
        out_type=jax.ShapeDtypeStruct((_NC * n, d), jnp.float32),
        mesh=mesh,
        compiler_params=pltpu.CompilerParams(needs_layout_passes=False),
        scratch_types=[
            pltpu.VMEM((_IBUF * bch,), jnp.int32),
            pltpu.VMEM((_IBUF * bch,), jnp.int32),
            [pltpu.VMEM((ch,), jnp.int32)] * _NBUF,
            [pltpu.VMEM((ch, d), jnp.float32)] * _NBUF,
            [pltpu.VMEM((ch,), jnp.float32)] * _NBUF,
            [pltpu.VMEM((ch,), jnp.float32)] * _NBUF,
            pltpu.VMEM_SHARED((n, d), jnp.float32),
            pltpu.SemaphoreType.DMA,
            [pltpu.SemaphoreType.DMA] * _NBUF,
            [pltpu.SemaphoreType.DMA] * _NBUF,
        ],
    )
    def k(s1_hbm, s2_hbm, row_hbm, col_hbm, x_hbm, out_hbm,
          rowb, colb, colv, xbufs, s1g, s2g, acc, bsem, gsems, ssems):
        cid = lax.axis_index("c")
        sid = lax.axis_index("s")
        wid = cid * _NS + sid
        ebase = wid * epw

        def blk_copy(b, bs, size):
            off = ebase + b * bch
            pltpu.async_copy(row_hbm.at[pl.ds(off, size)],
                             rowb.at[pl.ds(bs * bch, size)], bsem)
            pltpu.async_copy(col_hbm.at[pl.ds(off, size)],
                             colb.at[pl.ds(bs * bch, size)], bsem)

        def blk_drain(size):
            pltpu.make_async_copy(row_hbm.at[pl.ds(0, size)],
                                  rowb.at[pl.ds(0, size)], bsem).wait()
            pltpu.make_async_copy(col_hbm.at[pl.ds(0, size)],
                                  colb.at[pl.ds(0, size)], bsem).wait()

        def gathers_start(bs, kk, s):
            ridx = rowb.at[pl.ds(bs * bch + kk * ch, ch)]
            cidx = colb.at[pl.ds(bs * bch + kk * ch, ch)]
            pltpu.async_copy(x_hbm.at[ridx], xbufs[s], gsems[s])
            pltpu.async_copy(s1_hbm.at[ridx], s1g[s], gsems[s])
            pltpu.async_copy(s2_hbm.at[cidx], s2g[s], gsems[s])

        def gathers_drain(s):
            pltpu.make_async_copy(x_hbm.at[pl.ds(0, ch)], xbufs[s],
                                  gsems[s]).wait()
            pltpu.make_async_copy(s1_hbm.at[pl.ds(0, ch)], s1g[s],
                                  gsems[s]).wait()
            pltpu.make_async_copy(s2_hbm.at[pl.ds(0, ch)], s2g[s],
                                  gsems[s]).wait()

        def scatter_start(s):
            pltpu.async_copy(xbufs[s], acc.at[colv[s]], ssems[s], add=True)

        def scatter_drain(s):
            pltpu.make_async_copy(x_hbm.at[pl.ds(0, ch)], xbufs[s],
                                  ssems[s]).wait()

        def compute(bs, kk, s):
            xr, g1, g2 = xbufs[s], s1g[s], s2g[s]
            for j in range(ch // _L):
                colv[s][pl.ds(j * _L, _L)] = (
                    colb[pl.ds(bs * bch + kk * ch + j * _L, _L)])
            for j in range(ch // _L):
                v = g1[pl.ds(j * _L, _L)] + g2[pl.ds(j * _L, _L)]
                g1[pl.ds(j * _L, _L)] = 1.0 / (1.0 + jnp.exp(-v))

            def mul8(q, vidx):
                i0 = q * 8
                gis = []
                for r in range(8):
                    gis.append(plsc.load_gather(g1, [vidx]))
                    vidx = vidx + 1
                for r in range(8):
                    for j in range(d // _L):
                        xr[i0 + r, pl.ds(j * _L, _L)] = (
                            xr[i0 + r, pl.ds(j * _L, _L)] * gis[r])
                return vidx

            lax.fori_loop(0, ch // 8, mul8, jnp.zeros((_L,), jnp.int32))

        blk_copy(0, 0, bch)
        blk_copy(1, 1, bch)
        blk_drain(bch)
        gathers_start(0, 0, 0)
        gathers_start(0, 1, 1)

        zero16 = jnp.zeros((_L,), jnp.float32)
        zsrc = xbufs[2]

        def zrow(i, carry):
            for j in range(d // _L):
                zsrc[i, pl.ds(j * _L, _L)] = zero16
            return carry

        lax.fori_loop(0, ch, zrow, 0)

        def zblk(t, carry):
            blk = sid + t * _NS

            @pl.when(blk < nblk)
            def _():
                pltpu.async_copy(zsrc, acc.at[pl.ds(blk * br, br)], ssems[0])

            return carry

        def zdrain(t, carry):
            blk = sid + t * _NS

            @pl.when(blk < nblk)
            def _():
                pltpu.make_async_copy(zsrc, acc.at[pl.ds(0, br)],
                                      ssems[0]).wait()

            return carry

        lax.fori_loop(0, tpb, zblk, 0)
        lax.fori_loop(0, tpb, zdrain, 0)
        plsc.subcore_barrier()

        def step(t, carry):
            bs0 = lax.rem(t, _IBUF)
            bs1 = lax.rem(t + 1, _IBUF)
            bs2 = lax.rem(t + 2, _IBUF)
            for i in range(_NBUF):
                c = t * _NBUF + i
                sp2 = (i + 2) % _NBUF

                if i >= 2:
                    scatter_drain(i - 2)
                else:

                    @pl.when(t > 0)
                    def _():
                        scatter_drain((i - 2) % _NBUF)

                if i == 2:

                    @pl.when(t + 1 < nsteps)
                    def _():
                        blk_drain(bch)

                    if tail:

                        @pl.when(t + 1 == nsteps)
                        def _():
                            blk_drain(ltail)

                    @pl.when(t + 2 < nsteps)
                    def _():
                        blk_copy(t + 2, bs2, bch)

                    if tail:

                        @pl.when(t + 2 == nsteps)
                        def _():
                            blk_copy(t + 2, bs2, ltail)

                bsg = bs0 if i < 2 else bs1

                @pl.when(c + 2 < nch)
                def _():
                    gathers_start(bsg, (i + 2) % _NBUF, sp2)

                gathers_drain(i)
                compute(bs0, i, i)
                scatter_start(i)
            return carry

        lax.fori_loop(0, nsteps, step, 0)

        pending = [(_NBUF - 2) % _NBUF, (_NBUF - 1) % _NBUF]
        for c in tail:
            s = c % _NBUF
            gathers_drain(s)
            compute(nsteps % _IBUF, c % _NBUF, s)
            scatter_start(s)
            pending.append(s)
        for s in pending:
            scatter_drain(s)

        plsc.subcore_barrier()

        def eblk(t, carry):
            blk = sid + t * _NS

            @pl.when(blk < nblk)
            def _():
                pltpu.async_copy(acc.at[pl.ds(blk * br, br)],
                                 out_hbm.at[pl.ds(cid * n + blk * br, br)],
                                 bsem)

            return carry

        def edrain(t, carry):
            blk = sid + t * _NS

            @pl.when(blk < nblk)
            def _():
                pltpu.make_async_copy(acc.at[pl.ds(0, br)],
                                      out_hbm.at[pl.ds(0, br)], bsem).wait()

            return carry

        lax.fori_loop(0, tpb, eblk, 0)
        lax.fori_loop(0, tpb, edrain, 0)

    return k(s1, s2, row, col, x)


@functools.partial(jax.jit, static_argnames=("n", "d"))
def _combine(parts, *, n, d):
    def body(p_ref, o_ref):
        o_ref[...] = p_ref[0] + p_ref[1]

    return pl.pallas_call(
        body,
        out_shape=jax.ShapeDtypeStruct((n, d), jnp.float32),
    )(parts)


@jax.jit
def kernel(x, embed, edge_index, new_edge_index, label, tmp, W, b):
    n, d = x.shape
    e = edge_index.shape[1]
    row = edge_index[0].astype(jnp.int32)
    col = edge_index[1].astype(jnp.int32)
    w = W.astype(jnp.float32).reshape(2 * d)
    wpad = jnp.zeros((d, 8), jnp.float32)
    wpad = wpad.at[:, 0].set(w[:d]).at[:, 1].set(w[d:])

    s8 = _scores(wpad, embed.astype(jnp.float32), b.astype(jnp.float32),
                 n=n, d=d)
    parts = _sc_edge_aggregate(s8[0], s8[1], row, col,
                               x.astype(jnp.float32), n=n, d=d, e=e)
    return _combine(parts.reshape(_NC, n, d), n=n, d=d)
